# baseline (speedup 1.0000x reference)
import functools

import jax
import jax.numpy as jnp
from jax import lax
from jax.experimental import pallas as pl
from jax.experimental.pallas import tpu as pltpu
from jax.experimental.pallas import tpu_sc as plsc


def _tree_sum(vals):
    while len(vals) > 1:
        nxt = [vals[i] + vals[i + 1] for i in range(0, len(vals) - 1, 2)]
        if len(vals) % 2:
            nxt.append(vals[-1])
        vals = nxt
    return vals[0]


def _conv1_body(x_ref, w_ref, b_ref, h_ref, *, T, K, pad):
    b = b_ref[...]
    xb = {}
    for t in range(T):
        acc = None
        for k in range(K):
            tau = t - pad + k
            if tau < 0:
                continue
            if tau not in xb:
                xb[tau] = x_ref[:, tau, :].astype(jnp.bfloat16)
            term = jnp.dot(xb[tau], w_ref[k],
                           preferred_element_type=jnp.float32)
            acc = term if acc is None else acc + term
        h_ref[t] = jnp.maximum(acc + b, 0.0).astype(jnp.bfloat16)


def _conv1(x, w1t, b1r, nb):
    n, t, fin = x.shape
    k, _, fout = w1t.shape
    return pl.pallas_call(
        functools.partial(_conv1_body, T=t, K=k, pad=k - 1),
        grid=(n // nb,),
        in_specs=[
            pl.BlockSpec((nb, t, fin), lambda i: (i, 0, 0)),
            pl.BlockSpec((k, fin, fout), lambda i: (0, 0, 0)),
            pl.BlockSpec((1, fout), lambda i: (0, 0)),
        ],
        out_specs=pl.BlockSpec((t, nb, fout), lambda i: (0, i, 0)),
        out_shape=jax.ShapeDtypeStruct((t, n, fout), jnp.bfloat16),
    )(x, w1t, b1r)


def _make_gather_mean(TN, F, T, N, S):
    info = plsc.get_sparse_core_info()
    NC, NS = info.num_cores, info.num_subcores
    TPC = T // NC
    PPW = N // NS
    CP = 80 // S
    CI = CP * S
    NCH = PPW // CP
    FP = 125
    FCH = FP // CP
    NBUF = 8
    NFULL = (NCH // NBUF) * NBUF
    SW = PPW + (-PPW) % 8
    assert T % NC == 0 and N % NS == 0 and PPW % CP == 0
    assert PPW % FP == 0 and FP % CP == 0 and F % 32 == 0
    assert N % 8 == 0 and S == 16 and NCH - NFULL < NBUF - 1

    mesh = plsc.VectorSubcoreMesh(core_axis_name="c", subcore_axis_name="s")

    @functools.partial(
        pl.kernel, mesh=mesh,
        compiler_params=pltpu.CompilerParams(use_tc_tiling_on_sc=False,
                                             needs_layout_passes=False),
        out_type=jax.ShapeDtypeStruct((TN // FP, FP, F), jnp.bfloat16),
        scratch_types=[
            pltpu.VMEM((S, SW), jnp.int32),
            pltpu.VMEM((NCH, CI), jnp.int32),
            pltpu.VMEM((NBUF, CI, F), jnp.bfloat16),
            pltpu.VMEM((FP, F), jnp.bfloat16),
            pltpu.VMEM_SHARED((N, F), jnp.bfloat16),
            [pltpu.SemaphoreType.DMA] * NBUF,
            pltpu.SemaphoreType.DMA,
        ],
    )
    def gm(h_hbm, adj_hbm, out_hbm, slab_v, idx_v, rows_v, out_v, table, sems,
           sem_t):
        cid = lax.axis_index("c")
        sid = lax.axis_index("s")
        col0 = sid * PPW
        astart = (col0 // 8) * 8
        off = col0 - astart
        lanes = lax.iota(jnp.int32, 16)
        inv = jnp.bfloat16(1.0 / S)

        def start(c, b):
            pltpu.make_async_copy(table.at[idx_v.at[c]], rows_v.at[b],
                                  sems[b]).start()

        def wait(c, b):
            pltpu.make_async_copy(table.at[idx_v.at[c]], rows_v.at[b],
                                  sems[b]).wait()

        def accum(c, b):
            for p in range(CP):
                orow = (c % FCH) * CP + p
                for k in range(F // 32):
                    sl = pl.ds(k * 32, 32)
                    acc = _tree_sum([rows_v[b, p * S + r, sl] for r in range(S)])
                    out_v[orow, sl] = acc * inv

        def flush(c, t):
            @pl.when(c % FCH == FCH - 1)
            def _():
                fid = t * (N // FP) + sid * (PPW // FP) + c // FCH
                pltpu.sync_copy(out_v, out_hbm.at[fid])

        def phase(ph, carry):
            t = cid * TPC + ph
            plsc.subcore_barrier()
            tcopy = pltpu.make_async_copy(h_hbm.at[t, pl.ds(col0, PPW)],
                                          table.at[pl.ds(col0, PPW)], sem_t)
            tcopy.start()
            pltpu.sync_copy(adj_hbm.at[t, :, pl.ds(astart, SW)], slab_v)

            def prep(j, carry2):
                col = plsc.load_gather(slab_v, [lanes, lanes * 0 + (off + j)])
                idx_v[j // CP, pl.ds((j % CP) * S, S)] = col
                return carry2

            lax.fori_loop(0, PPW, prep, 0)
            tcopy.wait()
            plsc.subcore_barrier()

            for b in range(NBUF - 1):
                start(b, b)

            def body(i, carry2):
                for b in range(NBUF):
                    c = NBUF * i + b

                    @pl.when(c + NBUF - 1 < NCH)
                    def _next():
                        start(c + NBUF - 1, (b + NBUF - 1) % NBUF)

                    wait(c, b)
                    accum(c, b)
                    flush(c, t)
                return carry2

            lax.fori_loop(0, NFULL // NBUF, body, 0)
            for c in range(NFULL, NCH):
                wait(c, c % NBUF)
                accum(c, c % NBUF)
                flush(c, t)
            return carry

        lax.fori_loop(0, TPC, phase, 0)

    return gm


def _fc_conv2_body(h_ref, a_ref, wh_ref, wa_ref, bfc_ref, w2_ref, b2_ref,
                   o_ref, *, T, K, pad):
    bfc = bfc_ref[...]
    b2 = b2_ref[...]
    fc = []
    for t in range(T):
        z = (jnp.dot(h_ref[t], wh_ref[...], preferred_element_type=jnp.float32)
             + jnp.dot(a_ref[t], wa_ref[...], preferred_element_type=jnp.float32)
             + bfc)
        fc.append(jnp.maximum(z, 0.0).astype(jnp.bfloat16))
    for t in range(T):
        acc = None
        for k in range(K):
            tau = t - pad + k
            if tau < 0:
                continue
            term = jnp.dot(fc[tau], w2_ref[k],
                           preferred_element_type=jnp.float32)
            acc = term if acc is None else acc + term
        o_ref[:, t, :] = jnp.maximum(acc + b2, 0.0)


def _fc_conv2(h, agg, wfch, wfca, bfcr, w2t, b2r, nb):
    t, n, fout = h.shape
    hid = wfch.shape[1]
    k = w2t.shape[0]
    return pl.pallas_call(
        functools.partial(_fc_conv2_body, T=t, K=k, pad=k - 1),
        grid=(n // nb,),
        in_specs=[
            pl.BlockSpec((t, nb, fout), lambda i: (0, i, 0)),
            pl.BlockSpec((t, nb, fout), lambda i: (0, i, 0)),
            pl.BlockSpec((fout, hid), lambda i: (0, 0)),
            pl.BlockSpec((fout, hid), lambda i: (0, 0)),
            pl.BlockSpec((1, hid), lambda i: (0, 0)),
            pl.BlockSpec((k, hid, fout), lambda i: (0, 0, 0)),
            pl.BlockSpec((1, fout), lambda i: (0, 0)),
        ],
        out_specs=pl.BlockSpec((nb, t, fout), lambda i: (i, 0, 0)),
        out_shape=jax.ShapeDtypeStruct((n, t, fout), jnp.float32),
    )(h, agg, wfch, wfca, bfcr, w2t, b2r)


def kernel(x, adj_matrices, num_samples, batch_nodes, W1, b1, Wfc, bfc, W2, b2):
    del num_samples, batch_nodes
    n, t, fin = x.shape
    fout, _, k = W1.shape
    s = adj_matrices.shape[2]
    hid = Wfc.shape[0]

    w1t = jnp.transpose(W1, (2, 1, 0)).astype(jnp.bfloat16)
    w2t = jnp.transpose(W2, (2, 1, 0)).astype(jnp.bfloat16)
    wfch = jnp.transpose(Wfc[:, :fout]).astype(jnp.bfloat16)
    wfca = jnp.transpose(Wfc[:, fout:]).astype(jnp.bfloat16)
    b1r = b1.reshape(1, fout)
    bfcr = bfc.reshape(1, hid)
    b2r = b2.reshape(1, fout)

    nb = 400
    h = _conv1(x, w1t, b1r, nb)

    adjp = jnp.transpose(adj_matrices, (0, 2, 1))
    gm = _make_gather_mean(t * n, fout, t, n, s)
    agg = gm(h, adjp)
    agg = agg.reshape(t * n, fout)

    return _fc_conv2(h, agg.reshape(t, n, fout), wfch, wfca, bfcr, w2t, b2r, nb)


# speedup vs baseline: 1.3062x; 1.3062x over previous
import functools

import jax
import jax.numpy as jnp
from jax import lax
from jax.experimental import pallas as pl
from jax.experimental.pallas import tpu as pltpu
from jax.experimental.pallas import tpu_sc as plsc


def _tree_sum(vals):
    while len(vals) > 1:
        nxt = [vals[i] + vals[i + 1] for i in range(0, len(vals) - 1, 2)]
        if len(vals) % 2:
            nxt.append(vals[-1])
        vals = nxt
    return vals[0]


def _conv1_body(x_ref, w_ref, b_ref, h_ref, *, T, K, pad):
    b = b_ref[...]
    xb = {}
    for t in range(T):
        acc = None
        for k in range(K):
            tau = t - pad + k
            if tau < 0:
                continue
            if tau not in xb:
                xb[tau] = x_ref[:, tau, :].astype(jnp.bfloat16)
            term = jnp.dot(xb[tau], w_ref[k],
                           preferred_element_type=jnp.float32)
            acc = term if acc is None else acc + term
        h_ref[t] = jnp.maximum(acc + b, 0.0).astype(jnp.bfloat16)


def _conv1(x, w1t, b1r, nb):
    n, t, fin = x.shape
    k, _, fout = w1t.shape
    return pl.pallas_call(
        functools.partial(_conv1_body, T=t, K=k, pad=k - 1),
        grid=(n // nb,),
        in_specs=[
            pl.BlockSpec((nb, t, fin), lambda i: (i, 0, 0)),
            pl.BlockSpec((k, fin, fout), lambda i: (0, 0, 0)),
            pl.BlockSpec((1, fout), lambda i: (0, 0)),
        ],
        out_specs=pl.BlockSpec((t, nb, fout), lambda i: (0, i, 0)),
        out_shape=jax.ShapeDtypeStruct((t, n, fout), jnp.bfloat16),
    )(x, w1t, b1r)


def _make_gather_mean(TN, F, T, N, S):
    info = plsc.get_sparse_core_info()
    NC, NS = info.num_cores, info.num_subcores
    TPC = T // NC
    PPW = N // NS
    CP = 80 // S
    CI = CP * S
    NCH = PPW // CP
    FP = 125
    FCH = FP // CP
    NBUF = 4
    NFULL = (NCH // NBUF) * NBUF
    SW = PPW + (-PPW) % 8
    assert T % NC == 0 and N % NS == 0 and PPW % CP == 0
    assert PPW % FP == 0 and FP % CP == 0 and F % 32 == 0
    assert N % 8 == 0 and S == 16 and NCH - NFULL < NBUF - 1

    mesh = plsc.VectorSubcoreMesh(core_axis_name="c", subcore_axis_name="s")

    @functools.partial(
        pl.kernel, mesh=mesh,
        compiler_params=pltpu.CompilerParams(use_tc_tiling_on_sc=False,
                                             needs_layout_passes=False),
        out_type=jax.ShapeDtypeStruct((TN // FP, FP, F), jnp.bfloat16),
        scratch_types=[
            pltpu.VMEM((S, SW), jnp.int32),
            pltpu.VMEM((NCH, CI), jnp.int32),
            pltpu.VMEM((NBUF, CI, F), jnp.bfloat16),
            pltpu.VMEM((FP, F), jnp.bfloat16),
            pltpu.VMEM_SHARED((N, F), jnp.bfloat16),
            [pltpu.SemaphoreType.DMA] * NBUF,
            pltpu.SemaphoreType.DMA,
        ],
    )
    def gm(h_hbm, adj_hbm, out_hbm, slab_v, idx_v, rows_v, out_v, table, sems,
           sem_t):
        cid = lax.axis_index("c")
        sid = lax.axis_index("s")
        col0 = sid * PPW
        astart = (col0 // 8) * 8
        off = col0 - astart
        lanes = lax.iota(jnp.int32, 16)
        inv = jnp.bfloat16(1.0 / S)

        def start(c, b):
            pltpu.make_async_copy(table.at[idx_v.at[c]], rows_v.at[b],
                                  sems[b]).start()

        def wait(c, b):
            pltpu.make_async_copy(table.at[idx_v.at[c]], rows_v.at[b],
                                  sems[b]).wait()

        def accum(c, b):
            for p in range(CP):
                orow = (c % FCH) * CP + p
                for k in range(F // 32):
                    sl = pl.ds(k * 32, 32)
                    acc = _tree_sum([rows_v[b, p * S + r, sl] for r in range(S)])
                    out_v[orow, sl] = acc * inv

        def flush(c, t):
            @pl.when(c % FCH == FCH - 1)
            def _():
                fid = t * (N // FP) + sid * (PPW // FP) + c // FCH
                pltpu.sync_copy(out_v, out_hbm.at[fid])

        def phase(ph, carry):
            t = cid * TPC + ph
            plsc.subcore_barrier()
            tcopy = pltpu.make_async_copy(h_hbm.at[t, pl.ds(col0, PPW)],
                                          table.at[pl.ds(col0, PPW)], sem_t)
            tcopy.start()
            pltpu.sync_copy(adj_hbm.at[t, :, pl.ds(astart, SW)], slab_v)

            def prep(j, carry2):
                col = plsc.load_gather(slab_v, [lanes, lanes * 0 + (off + j)])
                idx_v[j // CP, pl.ds((j % CP) * S, S)] = col
                return carry2

            lax.fori_loop(0, PPW, prep, 0)
            tcopy.wait()
            plsc.subcore_barrier()

            for b in range(NBUF - 1):
                start(b, b)

            def body(i, carry2):
                for b in range(NBUF):
                    c = NBUF * i + b

                    @pl.when(c + NBUF - 1 < NCH)
                    def _next():
                        start(c + NBUF - 1, (b + NBUF - 1) % NBUF)

                    wait(c, b)
                    accum(c, b)
                    flush(c, t)
                return carry2

            lax.fori_loop(0, NFULL // NBUF, body, 0)
            for c in range(NFULL, NCH):
                wait(c, c % NBUF)
                accum(c, c % NBUF)
                flush(c, t)
            return carry

        lax.fori_loop(0, TPC, phase, 0)

    return gm


def _fc_conv2_body(h_ref, a_ref, wh_ref, wa_ref, bfc_ref, w2_ref, b2_ref,
                   o_ref, *, T, K, pad):
    bfc = bfc_ref[...]
    b2 = b2_ref[...]
    fc = []
    for t in range(T):
        z = (jnp.dot(h_ref[t], wh_ref[...], preferred_element_type=jnp.float32)
             + jnp.dot(a_ref[t], wa_ref[...], preferred_element_type=jnp.float32)
             + bfc)
        fc.append(jnp.maximum(z, 0.0).astype(jnp.bfloat16))
    for t in range(T):
        acc = None
        for k in range(K):
            tau = t - pad + k
            if tau < 0:
                continue
            term = jnp.dot(fc[tau], w2_ref[k],
                           preferred_element_type=jnp.float32)
            acc = term if acc is None else acc + term
        o_ref[:, t, :] = jnp.maximum(acc + b2, 0.0)


def _fc_conv2(h, agg, wfch, wfca, bfcr, w2t, b2r, nb):
    t, n, fout = h.shape
    hid = wfch.shape[1]
    k = w2t.shape[0]
    return pl.pallas_call(
        functools.partial(_fc_conv2_body, T=t, K=k, pad=k - 1),
        grid=(n // nb,),
        in_specs=[
            pl.BlockSpec((t, nb, fout), lambda i: (0, i, 0)),
            pl.BlockSpec((t, nb, fout), lambda i: (0, i, 0)),
            pl.BlockSpec((fout, hid), lambda i: (0, 0)),
            pl.BlockSpec((fout, hid), lambda i: (0, 0)),
            pl.BlockSpec((1, hid), lambda i: (0, 0)),
            pl.BlockSpec((k, hid, fout), lambda i: (0, 0, 0)),
            pl.BlockSpec((1, fout), lambda i: (0, 0)),
        ],
        out_specs=pl.BlockSpec((nb, t, fout), lambda i: (i, 0, 0)),
        out_shape=jax.ShapeDtypeStruct((n, t, fout), jnp.float32),
    )(h, agg, wfch, wfca, bfcr, w2t, b2r)


def kernel(x, adj_matrices, num_samples, batch_nodes, W1, b1, Wfc, bfc, W2, b2):
    del num_samples, batch_nodes
    n, t, fin = x.shape
    fout, _, k = W1.shape
    s = adj_matrices.shape[2]
    hid = Wfc.shape[0]

    w1t = jnp.transpose(W1, (2, 1, 0)).astype(jnp.bfloat16)
    w2t = jnp.transpose(W2, (2, 1, 0)).astype(jnp.bfloat16)
    wfch = jnp.transpose(Wfc[:, :fout]).astype(jnp.bfloat16)
    wfca = jnp.transpose(Wfc[:, fout:]).astype(jnp.bfloat16)
    b1r = b1.reshape(1, fout)
    bfcr = bfc.reshape(1, hid)
    b2r = b2.reshape(1, fout)

    nb = 400
    h = _conv1(x, w1t, b1r, nb)

    adjp = jnp.transpose(adj_matrices, (0, 2, 1))
    gm = _make_gather_mean(t * n, fout, t, n, s)
    agg = gm(h, adjp)
    agg = agg.reshape(t * n, fout)

    return _fc_conv2(h, agg.reshape(t, n, fout), wfch, wfca, bfcr, w2t, b2r, nb)
